# baseline (speedup 1.0000x reference)
import functools

import jax
import jax.numpy as jnp
from jax import lax
from jax.experimental import pallas as pl
from jax.experimental.pallas import tpu as pltpu
from jax.experimental.pallas import tpu_sc as plsc

EMBED_DIM = 64
BBLK = 128


def _make_gather(n_b: int, n_s: int):
    info = plsc.get_sparse_core_info()
    nw = info.num_cores * info.num_subcores
    lanes = info.num_lanes
    assert n_b == nw * BBLK and n_s % 2 == 0
    mesh = plsc.VectorSubcoreMesh(core_axis_name="c", subcore_axis_name="s")

    @functools.partial(
        pl.kernel,
        mesh=mesh,
        out_type=jax.ShapeDtypeStruct((n_s, EMBED_DIM, n_b), jnp.float32),
        scratch_types=[
            pltpu.VMEM((n_s, BBLK), jnp.int32),
            pltpu.VMEM((BBLK, EMBED_DIM), jnp.float32),
            pltpu.VMEM((BBLK, EMBED_DIM), jnp.float32),
            pltpu.VMEM((EMBED_DIM, BBLK), jnp.float32),
            pltpu.VMEM((EMBED_DIM, BBLK), jnp.float32),
            pltpu.SemaphoreType.DMA,
            pltpu.SemaphoreType.DMA,
        ],
        compiler_params=pltpu.CompilerParams(
            use_tc_tiling_on_sc=False, needs_layout_passes=False
        ),
    )
    def gather_kernel(ids_hbm, table_hbm, out_hbm, idx_v, rows0, rows1,
                      rowst0, rowst1, gsem, wsem):
        wid = lax.axis_index("s") * info.num_cores + lax.axis_index("c")
        b0 = wid * BBLK
        rows = (rows0, rows1)
        rowst = (rowst0, rowst1)

        pltpu.sync_copy(ids_hbm.at[:, pl.ds(b0, BBLK)], idx_v)

        lane = lax.iota(jnp.int32, lanes)

        def fetch(g, b):
            pltpu.async_copy(table_hbm.at[idx_v.at[g]], rows[b], gsem)

        def wait_gather(g, b):
            pltpu.make_async_copy(
                table_hbm.at[idx_v.at[g]], rows[b], gsem
            ).wait()

        def transpose(b):
            src = rows[b]
            dst = rowst[b]
            for bb in range(BBLK // lanes):
                row_idx = bb * lanes + lane
                for d in range(EMBED_DIM):
                    col_idx = jnp.full((lanes,), d, jnp.int32)
                    dst[d, pl.ds(bb * lanes, lanes)] = plsc.load_gather(
                        src, [row_idx, col_idx]
                    )

        def start_wb(g, b):
            pltpu.async_copy(
                rowst[b], out_hbm.at[g, :, pl.ds(b0, BBLK)], wsem
            )

        def wait_wb(g, b):
            pltpu.make_async_copy(
                rowst[b], out_hbm.at[g, :, pl.ds(b0, BBLK)], wsem
            ).wait()

        fetch(0, 0)
        fetch(1, 1)
        wait_gather(0, 0)
        transpose(0)
        start_wb(0, 0)

        def pair(u, carry):
            g0 = 2 * u
            wait_wb(g0 - 2, 0)
            fetch(g0, 0)
            wait_gather(g0 - 1, 1)
            transpose(1)
            start_wb(g0 - 1, 1)
            wait_wb(g0 - 1, 1)
            fetch(g0 + 1, 1)
            wait_gather(g0, 0)
            transpose(0)
            start_wb(g0, 0)
            return carry

        lax.fori_loop(1, n_s // 2, pair, 0)

        wait_gather(n_s - 1, 1)
        transpose(1)
        start_wb(n_s - 1, 1)
        wait_wb(n_s - 2, 0)
        wait_wb(n_s - 1, 1)

    return gather_kernel


def kernel(ids, ori_weight, think_weight):
    table = jnp.concatenate([ori_weight, think_weight], axis=0)
    n_b, n_s = ids.shape
    out_t = _make_gather(n_b, n_s)(ids.T, table)
    return out_t.transpose(2, 0, 1)


# speedup vs baseline: 2.1539x; 2.1539x over previous
import functools

import jax
import jax.numpy as jnp
from jax import lax
from jax.experimental import pallas as pl
from jax.experimental.pallas import tpu as pltpu
from jax.experimental.pallas import tpu_sc as plsc

EMBED_DIM = 64


def _make_gather(n_b: int, n_s: int):
    info = plsc.get_sparse_core_info()
    nw = info.num_cores * info.num_subcores
    assert n_b % (2 * nw) == 0
    b_per_w = n_b // nw
    mesh = plsc.VectorSubcoreMesh(core_axis_name="c", subcore_axis_name="s")

    @functools.partial(
        pl.kernel,
        mesh=mesh,
        out_type=jax.ShapeDtypeStruct((n_b, n_s, EMBED_DIM), jnp.float32),
        scratch_types=[
            pltpu.VMEM((b_per_w, n_s), jnp.int32),
            pltpu.VMEM((2, n_s, EMBED_DIM), jnp.float32),
            pltpu.SemaphoreType.DMA,
            pltpu.SemaphoreType.DMA,
        ],
        compiler_params=pltpu.CompilerParams(use_tc_tiling_on_sc=False),
    )
    def gather_kernel(ids_hbm, table_hbm, out_hbm, idx_v, rows_v, gsem, wsem):
        wid = lax.axis_index("s") * info.num_cores + lax.axis_index("c")
        base = wid * b_per_w

        pltpu.sync_copy(ids_hbm.at[wid], idx_v)

        def fetch(g, b):
            pltpu.async_copy(table_hbm.at[idx_v.at[g]], rows_v.at[b], gsem)

        def wait_gather(g, b):
            pltpu.make_async_copy(
                table_hbm.at[idx_v.at[g]], rows_v.at[b], gsem
            ).wait()

        def start_wb(g, b):
            pltpu.async_copy(rows_v.at[b], out_hbm.at[base + g], wsem)

        def wait_wb(g, b):
            pltpu.make_async_copy(
                rows_v.at[b], out_hbm.at[base + g], wsem
            ).wait()

        fetch(0, 0)
        fetch(1, 1)
        wait_gather(0, 0)
        start_wb(0, 0)

        def pair(u, carry):
            g0 = 2 * u
            wait_wb(g0 - 2, 0)
            fetch(g0, 0)
            wait_gather(g0 - 1, 1)
            start_wb(g0 - 1, 1)
            wait_wb(g0 - 1, 1)
            fetch(g0 + 1, 1)
            wait_gather(g0, 0)
            start_wb(g0, 0)
            return carry

        lax.fori_loop(1, b_per_w // 2, pair, 0)

        wait_gather(b_per_w - 1, 1)
        start_wb(b_per_w - 1, 1)
        wait_wb(b_per_w - 2, 0)
        wait_wb(b_per_w - 1, 1)

    return gather_kernel


def kernel(ids, ori_weight, think_weight):
    table = jnp.concatenate([ori_weight, think_weight], axis=0)
    n_b, n_s = ids.shape
    info = plsc.get_sparse_core_info()
    nw = info.num_cores * info.num_subcores
    ids_w = ids.reshape(nw, n_b // nw, n_s)
    return _make_gather(n_b, n_s)(ids_w, table)
